# Initial kernel scaffold; baseline (speedup 1.0000x reference)
import jax
import jax.numpy as jnp
from jax.experimental import pallas as pl


def kernel(x, W_route, b_route, W_noise, b_noise):
    raise NotImplementedError("write your pallas kernel here")


# speedup vs baseline: 3.0115x; 3.0115x over previous
import functools

import jax
import jax.numpy as jnp
from jax.experimental import pallas as pl

_N_TOK = 32768
_N_EXP = 64
_TOP_K = 8
_BT = 256


def _router_block(x_ref, w_ref, b_ref, noise_ref, router_ref, idx_ref, probs_ref):
    acc = jnp.dot(x_ref[...], w_ref[...], preferred_element_type=jnp.float32)
    acc = acc + b_ref[...]
    logits = acc[:, :_N_EXP]
    noise_logits = acc[:, _N_EXP:]
    noisy = logits + noise_ref[...] * jax.nn.softplus(noise_logits)

    m = jnp.max(noisy, axis=-1, keepdims=True)
    e = jnp.exp(noisy - m)
    probs_ref[...] = e / jnp.sum(e, axis=-1, keepdims=True)

    lane = jax.lax.broadcasted_iota(jnp.int32, noisy.shape, 1)
    cur = noisy
    selected = jnp.zeros(noisy.shape, dtype=jnp.bool_)
    for k in range(_TOP_K):
        mk = jnp.max(cur, axis=-1, keepdims=True)
        idx_k = jnp.min(jnp.where(cur == mk, lane, _N_EXP), axis=-1, keepdims=True)
        chosen = lane == idx_k
        selected = selected | chosen
        cur = jnp.where(chosen, -jnp.inf, cur)
        idx_ref[:, k : k + 1] = idx_k

    es = jnp.where(selected, e, 0.0)
    router_ref[...] = es / jnp.sum(es, axis=-1, keepdims=True)


@functools.partial(jax.jit, static_argnums=())
def _run(x, w, b, noise):
    grid = (_N_TOK // _BT,)
    n_embed = x.shape[1]
    return pl.pallas_call(
        _router_block,
        grid=grid,
        in_specs=[
            pl.BlockSpec((_BT, n_embed), lambda i: (i, 0)),
            pl.BlockSpec((n_embed, 2 * _N_EXP), lambda i: (0, 0)),
            pl.BlockSpec((1, 2 * _N_EXP), lambda i: (0, 0)),
            pl.BlockSpec((_BT, _N_EXP), lambda i: (i, 0)),
        ],
        out_specs=[
            pl.BlockSpec((_BT, _N_EXP), lambda i: (i, 0)),
            pl.BlockSpec((_BT, _TOP_K), lambda i: (i, 0)),
            pl.BlockSpec((_BT, _N_EXP), lambda i: (i, 0)),
        ],
        out_shape=[
            jax.ShapeDtypeStruct((_N_TOK, _N_EXP), jnp.float32),
            jax.ShapeDtypeStruct((_N_TOK, _TOP_K), jnp.int32),
            jax.ShapeDtypeStruct((_N_TOK, _N_EXP), jnp.float32),
        ],
    )(x, w, b, noise)


def kernel(x, W_route, b_route, W_noise, b_noise):
    w = jnp.concatenate([W_route, W_noise], axis=1)
    b = jnp.concatenate([b_route, b_noise])[None, :]
    noise = jax.random.normal(
        jax.random.key(42), (x.shape[0], _N_EXP), dtype=jnp.float32
    )
    router_out, idx, full_probs = _run(x, w, b, noise)
    return (router_out, idx, full_probs)


# speedup vs baseline: 3.9814x; 1.3220x over previous
import functools

import jax
import jax.numpy as jnp
from jax.experimental import pallas as pl

_N_TOK = 32768
_N_EXP = 64
_TOP_K = 8
_BT = 512


def _router_block(x_ref, w_ref, b_ref, noise_ref, router_ref, idx_ref, probs_ref):
    acc = jnp.dot(x_ref[...], w_ref[...], preferred_element_type=jnp.float32)
    acc = acc + b_ref[...]
    logits = acc[:, :_N_EXP]
    noise_logits = acc[:, _N_EXP:]
    noisy = logits + noise_ref[...] * jax.nn.softplus(noise_logits)

    cur = noisy
    thr = []
    for _ in range(_TOP_K):
        mk = jnp.max(cur, axis=-1, keepdims=True)
        thr.append(mk)
        cur = jnp.where(cur == mk, -jnp.inf, cur)

    e = jnp.exp(noisy - thr[0])
    probs_ref[...] = e / jnp.sum(e, axis=-1, keepdims=True)

    es = jnp.where(noisy >= thr[_TOP_K - 1], e, 0.0)
    router_ref[...] = es / jnp.sum(es, axis=-1, keepdims=True)

    lane = jax.lax.broadcasted_iota(jnp.int32, noisy.shape, 1)
    idx_cols = [
        jnp.min(jnp.where(noisy == t, lane, _N_EXP), axis=-1, keepdims=True)
        for t in thr
    ]
    idx_ref[...] = jnp.concatenate(idx_cols, axis=1)


@functools.partial(jax.jit, static_argnums=())
def _run(x, w, b, noise):
    grid = (_N_TOK // _BT,)
    n_embed = x.shape[1]
    return pl.pallas_call(
        _router_block,
        grid=grid,
        in_specs=[
            pl.BlockSpec((_BT, n_embed), lambda i: (i, 0)),
            pl.BlockSpec((n_embed, 2 * _N_EXP), lambda i: (0, 0)),
            pl.BlockSpec((1, 2 * _N_EXP), lambda i: (0, 0)),
            pl.BlockSpec((_BT, _N_EXP), lambda i: (i, 0)),
        ],
        out_specs=[
            pl.BlockSpec((_BT, _N_EXP), lambda i: (i, 0)),
            pl.BlockSpec((_BT, _TOP_K), lambda i: (i, 0)),
            pl.BlockSpec((_BT, _N_EXP), lambda i: (i, 0)),
        ],
        out_shape=[
            jax.ShapeDtypeStruct((_N_TOK, _N_EXP), jnp.float32),
            jax.ShapeDtypeStruct((_N_TOK, _TOP_K), jnp.int32),
            jax.ShapeDtypeStruct((_N_TOK, _N_EXP), jnp.float32),
        ],
    )(x, w, b, noise)


def kernel(x, W_route, b_route, W_noise, b_noise):
    w = jnp.concatenate([W_route, W_noise], axis=1)
    b = jnp.concatenate([b_route, b_noise])[None, :]
    noise = jax.random.normal(
        jax.random.key(42), (x.shape[0], _N_EXP), dtype=jnp.float32
    )
    router_out, idx, full_probs = _run(x, w, b, noise)
    return (router_out, idx, full_probs)


# speedup vs baseline: 4.1586x; 1.0445x over previous
import functools

import jax
import jax.numpy as jnp
from jax.experimental import pallas as pl

_N_TOK = 32768
_N_EXP = 64
_TOP_K = 8
_BT = 512


def _router_block(x_ref, w_ref, b_ref, noise_ref, router_ref, idx_ref, probs_ref):
    acc = jnp.dot(x_ref[...], w_ref[...], preferred_element_type=jnp.float32)
    acc = acc + b_ref[...]
    logits = acc[:, :_N_EXP]
    noise_logits = acc[:, _N_EXP:]
    noisy = logits + noise_ref[...] * jax.nn.softplus(noise_logits)

    cur = noisy
    thr = []
    for _ in range(_TOP_K):
        mk = jnp.max(cur, axis=-1, keepdims=True)
        thr.append(mk)
        cur = jnp.where(cur == mk, -jnp.inf, cur)

    e = jnp.exp(noisy - thr[0])
    probs_ref[...] = e / jnp.sum(e, axis=-1, keepdims=True)

    es = jnp.where(noisy >= thr[_TOP_K - 1], e, 0.0)
    router_ref[...] = es / jnp.sum(es, axis=-1, keepdims=True)

    lane = jax.lax.broadcasted_iota(jnp.int32, noisy.shape, 1).astype(jnp.float32)
    idx_cols = [
        jnp.min(jnp.where(noisy == t, lane, float(_N_EXP)), axis=-1, keepdims=True)
        for t in thr
    ]
    idx_ref[...] = jnp.concatenate(idx_cols, axis=1).astype(jnp.int32)


@functools.partial(jax.jit, static_argnums=())
def _run(x, w, b, noise):
    grid = (_N_TOK // _BT,)
    n_embed = x.shape[1]
    return pl.pallas_call(
        _router_block,
        grid=grid,
        in_specs=[
            pl.BlockSpec((_BT, n_embed), lambda i: (i, 0)),
            pl.BlockSpec((n_embed, 2 * _N_EXP), lambda i: (0, 0)),
            pl.BlockSpec((1, 2 * _N_EXP), lambda i: (0, 0)),
            pl.BlockSpec((_BT, _N_EXP), lambda i: (i, 0)),
        ],
        out_specs=[
            pl.BlockSpec((_BT, _N_EXP), lambda i: (i, 0)),
            pl.BlockSpec((_BT, _TOP_K), lambda i: (i, 0)),
            pl.BlockSpec((_BT, _N_EXP), lambda i: (i, 0)),
        ],
        out_shape=[
            jax.ShapeDtypeStruct((_N_TOK, _N_EXP), jnp.float32),
            jax.ShapeDtypeStruct((_N_TOK, _TOP_K), jnp.int32),
            jax.ShapeDtypeStruct((_N_TOK, _N_EXP), jnp.float32),
        ],
    )(x, w, b, noise)


def kernel(x, W_route, b_route, W_noise, b_noise):
    w = jnp.concatenate([W_route, W_noise], axis=1)
    b = jnp.concatenate([b_route, b_noise])[None, :]
    noise = jax.random.normal(
        jax.random.key(42), (x.shape[0], _N_EXP), dtype=jnp.float32
    )
    router_out, idx, full_probs = _run(x, w, b, noise)
    return (router_out, idx, full_probs)
